# Initial kernel scaffold; baseline (speedup 1.0000x reference)
import jax
import jax.numpy as jnp
from jax.experimental import pallas as pl


def kernel(zs, rows, cols, row_embedding, col_embedding):
    raise NotImplementedError("write your pallas kernel here")


# speedup vs baseline: 21.4139x; 21.4139x over previous
import functools

import jax
import jax.numpy as jnp
from jax import lax
from jax.experimental import pallas as pl
from jax.experimental.pallas import tpu as pltpu
from jax.experimental.pallas import tpu_sc as plsc

D = 64
TBL = 128
NC = 2
NS = 16
NW = NC * NS
CHUNK = 512
G = CHUNK // 128


def _table_body(rz_ref, cz_ref, re_ref, ce_ref, out_ref):
    ioz = lax.broadcasted_iota(jnp.int32, (TBL, TBL), 1)
    oh_r = (rz_ref[...] == ioz).astype(jnp.float32)
    oh_c = (cz_ref[...] == ioz).astype(jnp.float32)
    out_ref[...] = (
        jnp.dot(oh_r, re_ref[...], preferred_element_type=jnp.float32)
        + jnp.dot(oh_c, ce_ref[...], preferred_element_type=jnp.float32)
    )


@jax.jit
def _build_table(rz, cz, re_p, ce_p):
    return pl.pallas_call(
        _table_body,
        out_shape=jax.ShapeDtypeStruct((TBL, D), jnp.float32),
    )(rz, cz, re_p, ce_p)


def _make_gather(n_tokens):
    bpw = n_tokens // NW
    n_chunks = bpw // CHUNK
    rows_per_chunk = CHUNK // 128

    mesh = plsc.VectorSubcoreMesh(core_axis_name="c", subcore_axis_name="s")

    @functools.partial(
        pl.kernel,
        mesh=mesh,
        out_type=jax.ShapeDtypeStruct((n_tokens, D), jnp.float32),
        compiler_params=pltpu.CompilerParams(use_tc_tiling_on_sc=False),
        scratch_types=[
            pltpu.VMEM((G, 128), jnp.int32),
            pltpu.VMEM((CHUNK, D), jnp.float32),
            pltpu.SemaphoreType.DMA,
        ],
    )
    def gather(table_hbm, zs_hbm, out_hbm, idx_v, rows_v, sem):
        wid = lax.axis_index("s") * NC + lax.axis_index("c")
        tok_base = wid * bpw
        row_base = wid * (bpw // 128)

        def chunk_body(c, carry):
            pltpu.sync_copy(
                zs_hbm.at[pl.ds(row_base + c * rows_per_chunk, G)], idx_v
            )
            copies = []
            for j in range(G):
                copies.append(
                    pltpu.async_copy(
                        table_hbm.at[idx_v.at[j]],
                        rows_v.at[pl.ds(j * 128, 128)],
                        sem,
                    )
                )
            for cp in copies:
                cp.wait()
            pltpu.sync_copy(
                rows_v, out_hbm.at[pl.ds(tok_base + c * CHUNK, CHUNK)]
            )
            return carry

        lax.fori_loop(0, n_chunks, chunk_body, 0)

    return gather


def kernel(zs, rows, cols, row_embedding, col_embedding):
    b, s = zs.shape
    n = b * s
    rz = jnp.zeros((TBL,), jnp.int32).at[1 : 1 + rows.shape[0]].set(rows)
    cz = jnp.zeros((TBL,), jnp.int32).at[1 : 1 + cols.shape[0]].set(cols)
    re_p = jnp.zeros((TBL, D), jnp.float32).at[: row_embedding.shape[0]].set(
        row_embedding
    )
    ce_p = jnp.zeros((TBL, D), jnp.float32).at[: col_embedding.shape[0]].set(
        col_embedding
    )
    table = _build_table(rz.reshape(TBL, 1), cz.reshape(TBL, 1), re_p, ce_p)
    zs2 = zs.reshape(-1, 128)
    out = _make_gather(n)(table, zs2)
    return out.reshape(b, s, D)


# speedup vs baseline: 21.4527x; 1.0018x over previous
import functools

import jax
import jax.numpy as jnp
from jax import lax
from jax.experimental import pallas as pl
from jax.experimental.pallas import tpu as pltpu
from jax.experimental.pallas import tpu_sc as plsc

D = 64
TBL = 128
NC = 2
NS = 16
NW = NC * NS
CHUNK = 512
G = CHUNK // 128


def _table_body(rz_ref, cz_ref, re_ref, ce_ref, out_ref):
    ioz = lax.broadcasted_iota(jnp.int32, (TBL, TBL), 1)
    oh_r = (rz_ref[...] == ioz).astype(jnp.float32)
    oh_c = (cz_ref[...] == ioz).astype(jnp.float32)
    out_ref[...] = (
        jnp.dot(oh_r, re_ref[...], preferred_element_type=jnp.float32)
        + jnp.dot(oh_c, ce_ref[...], preferred_element_type=jnp.float32)
    )


@jax.jit
def _build_table(rz, cz, re_p, ce_p):
    return pl.pallas_call(
        _table_body,
        out_shape=jax.ShapeDtypeStruct((TBL, D), jnp.float32),
    )(rz, cz, re_p, ce_p)


IB = 4


def _make_gather(n_tokens):
    bpw = n_tokens // NW
    n_chunks = bpw // CHUNK
    n_blocks = n_chunks // IB
    ib_rows = IB * G

    mesh = plsc.VectorSubcoreMesh(core_axis_name="c", subcore_axis_name="s")

    @functools.partial(
        pl.kernel,
        mesh=mesh,
        out_type=jax.ShapeDtypeStruct((n_tokens, D), jnp.float32),
        compiler_params=pltpu.CompilerParams(use_tc_tiling_on_sc=False),
        scratch_types=[
            pltpu.VMEM((2, ib_rows, 128), jnp.int32),
            pltpu.VMEM((2, CHUNK, D), jnp.float32),
            pltpu.SemaphoreType.DMA,
            pltpu.SemaphoreType.DMA,
            pltpu.SemaphoreType.DMA,
        ],
    )
    def gather(table_hbm, zs_hbm, out_hbm, idx_v, rows_v, sem_i, sem_g, sem_w):
        wid = lax.axis_index("s") * NC + lax.axis_index("c")
        tok_base = wid * bpw
        row_base = wid * (bpw // 128)

        pltpu.async_copy(
            zs_hbm.at[pl.ds(row_base, ib_rows)], idx_v.at[0], sem_i
        )

        def block_body(b, carry):
            slot = lax.rem(b, 2)
            pltpu.make_async_copy(
                zs_hbm.at[pl.ds(row_base, ib_rows)], idx_v.at[slot], sem_i
            ).wait()

            @pl.when(b + 1 < n_blocks)
            def _():
                pltpu.async_copy(
                    zs_hbm.at[pl.ds(row_base + (b + 1) * ib_rows, ib_rows)],
                    idx_v.at[1 - slot],
                    sem_i,
                )

            for i in range(IB):
                c = b * IB + i
                buf = lax.rem(c, 2)

                @pl.when(c >= 2)
                def _():
                    pltpu.make_async_copy(
                        rows_v.at[0],
                        out_hbm.at[pl.ds(tok_base, CHUNK)],
                        sem_w,
                    ).wait()

                cps = [
                    pltpu.async_copy(
                        table_hbm.at[idx_v.at[slot, i * G + j]],
                        rows_v.at[buf, pl.ds(j * 128, 128)],
                        sem_g,
                    )
                    for j in range(G)
                ]
                for cp in cps:
                    cp.wait()
                pltpu.async_copy(
                    rows_v.at[buf],
                    out_hbm.at[pl.ds(tok_base + c * CHUNK, CHUNK)],
                    sem_w,
                )
            return carry

        lax.fori_loop(0, n_blocks, block_body, 0)
        for _ in range(2):
            pltpu.make_async_copy(
                rows_v.at[0], out_hbm.at[pl.ds(tok_base, CHUNK)], sem_w
            ).wait()

    return gather


def kernel(zs, rows, cols, row_embedding, col_embedding):
    b, s = zs.shape
    n = b * s
    rz = jnp.zeros((TBL,), jnp.int32).at[1 : 1 + rows.shape[0]].set(rows)
    cz = jnp.zeros((TBL,), jnp.int32).at[1 : 1 + cols.shape[0]].set(cols)
    re_p = jnp.zeros((TBL, D), jnp.float32).at[: row_embedding.shape[0]].set(
        row_embedding
    )
    ce_p = jnp.zeros((TBL, D), jnp.float32).at[: col_embedding.shape[0]].set(
        col_embedding
    )
    table = _build_table(rz.reshape(TBL, 1), cz.reshape(TBL, 1), re_p, ce_p)
    zs2 = zs.reshape(-1, 128)
    out = _make_gather(n)(table, zs2)
    return out.reshape(b, s, D)


# speedup vs baseline: 49.5324x; 2.3089x over previous
import functools

import jax
import jax.numpy as jnp
from jax import lax
from jax.experimental import pallas as pl
from jax.experimental.pallas import tpu as pltpu
from jax.experimental.pallas import tpu_sc as plsc

D = 64
TBL = 128
NC = 2
NS = 16
NW = NC * NS
SBLK = 8


def _table_body(rz_ref, cz_ref, re_ref, ce_ref, out_ref):
    rz = rz_ref[...]
    cz = cz_ref[...]
    acc = jnp.zeros((D, TBL), jnp.float32)
    for r in range(9):
        acc = acc + jnp.where(rz == r, re_ref[:, r][:, None], 0.0)
    for c in range(18):
        acc = acc + jnp.where(cz == c, ce_ref[:, c][:, None], 0.0)
    out_ref[...] = acc


@jax.jit
def _build_table(rz, cz, re_t, ce_t):
    return pl.pallas_call(
        _table_body,
        out_shape=jax.ShapeDtypeStruct((D, TBL), jnp.float32),
    )(rz, cz, re_t, ce_t)


def _make_gather(batch, seq):
    bpw = batch // NW
    n_sblk = seq // SBLK

    mesh = plsc.VectorSubcoreMesh(core_axis_name="c", subcore_axis_name="s")

    @functools.partial(
        pl.kernel,
        mesh=mesh,
        out_type=jax.ShapeDtypeStruct((seq, D, batch), jnp.float32),
        compiler_params=pltpu.CompilerParams(needs_layout_passes=False),
        scratch_types=[
            pltpu.VMEM((D, TBL), jnp.float32),
            pltpu.VMEM((2, SBLK, bpw), jnp.int32),
            pltpu.VMEM((2, D, bpw), jnp.float32),
            pltpu.SemaphoreType.DMA,
            pltpu.SemaphoreType.DMA,
        ],
    )
    def gather(table_hbm, zst_hbm, out_hbm, tt_v, zs_v, out_v, sem_z, sem_w):
        wid = lax.axis_index("s") * NC + lax.axis_index("c")
        b0 = wid * bpw

        pltpu.sync_copy(table_hbm, tt_v)
        pltpu.async_copy(
            zst_hbm.at[pl.ds(0, SBLK), pl.ds(b0, bpw)], zs_v.at[0], sem_z
        )

        def sblk_body(blk, carry):
            slot = lax.rem(blk, 2)
            pltpu.make_async_copy(
                zst_hbm.at[pl.ds(0, SBLK), pl.ds(b0, bpw)],
                zs_v.at[slot],
                sem_z,
            ).wait()

            @pl.when(blk + 1 < n_sblk)
            def _():
                pltpu.async_copy(
                    zst_hbm.at[pl.ds((blk + 1) * SBLK, SBLK), pl.ds(b0, bpw)],
                    zs_v.at[1 - slot],
                    sem_z,
                )

            for i in range(SBLK):
                s = blk * SBLK + i
                buf = lax.rem(s, 2)

                @pl.when(s >= 2)
                def _():
                    pltpu.make_async_copy(
                        out_v.at[0],
                        out_hbm.at[0, :, pl.ds(b0, bpw)],
                        sem_w,
                    ).wait()

                def k_body(k, kcarry):
                    z16 = zs_v[slot, i, pl.ds(16 * k, 16)]
                    for d in range(D):
                        out_v[buf, d, pl.ds(16 * k, 16)] = plsc.load_gather(
                            tt_v.at[d], [z16]
                        )
                    return kcarry

                lax.fori_loop(0, bpw // 16, k_body, 0)
                pltpu.async_copy(
                    out_v.at[buf], out_hbm.at[s, :, pl.ds(b0, bpw)], sem_w
                )
            return carry

        lax.fori_loop(0, n_sblk, sblk_body, 0)
        for _ in range(2):
            pltpu.make_async_copy(
                out_v.at[0], out_hbm.at[0, :, pl.ds(b0, bpw)], sem_w
            ).wait()

    return gather


def kernel(zs, rows, cols, row_embedding, col_embedding):
    batch, seq = zs.shape
    rz = jnp.zeros((TBL,), jnp.int32).at[1 : 1 + rows.shape[0]].set(rows)
    cz = jnp.zeros((TBL,), jnp.int32).at[1 : 1 + cols.shape[0]].set(cols)
    table = _build_table(
        rz.reshape(1, TBL), cz.reshape(1, TBL), row_embedding.T, col_embedding.T
    )
    out_t = _make_gather(batch, seq)(table, zs.T)
    return jnp.transpose(out_t, (2, 0, 1))


# speedup vs baseline: 115.9133x; 2.3401x over previous
import functools

import jax
import jax.numpy as jnp
from jax import lax
from jax.experimental import pallas as pl
from jax.experimental.pallas import tpu as pltpu
from jax.experimental.pallas import tpu_sc as plsc

D = 64
TBL = 128
NC = 2
NS = 16
NW = NC * NS
SBLK = 8


def _table_body(rz_ref, cz_ref, re_ref, ce_ref, out_ref):
    rz = rz_ref[...]
    cz = cz_ref[...]
    acc = jnp.zeros((D, TBL), jnp.float32)
    for r in range(9):
        acc = acc + jnp.where(rz == r, re_ref[:, r][:, None], 0.0)
    for c in range(18):
        acc = acc + jnp.where(cz == c, ce_ref[:, c][:, None], 0.0)
    out_ref[...] = acc


@jax.jit
def _build_table(rz, cz, re_t, ce_t):
    return pl.pallas_call(
        _table_body,
        out_shape=jax.ShapeDtypeStruct((D, TBL), jnp.float32),
    )(rz, cz, re_t, ce_t)


def _make_gather(batch, seq):
    bpw = batch // NW
    n_sblk = seq // SBLK

    mesh = plsc.VectorSubcoreMesh(core_axis_name="c", subcore_axis_name="s")

    @functools.partial(
        pl.kernel,
        mesh=mesh,
        out_type=jax.ShapeDtypeStruct((seq, D, batch), jnp.float32),
        compiler_params=pltpu.CompilerParams(needs_layout_passes=False),
        scratch_types=[
            pltpu.VMEM((D, TBL), jnp.float32),
            pltpu.VMEM((2, SBLK, bpw), jnp.int32),
            pltpu.VMEM((2, D, bpw), jnp.float32),
            pltpu.SemaphoreType.DMA,
            pltpu.SemaphoreType.DMA,
        ],
    )
    def gather(table_hbm, zst_hbm, out_hbm, tt_v, zs_v, out_v, sem_z, sem_w):
        wid = lax.axis_index("s") * NC + lax.axis_index("c")
        b0 = wid * bpw

        pltpu.sync_copy(table_hbm, tt_v)
        pltpu.async_copy(
            zst_hbm.at[pl.ds(0, SBLK), pl.ds(b0, bpw)], zs_v.at[0], sem_z
        )

        def sblk_body(blk, carry):
            slot = lax.rem(blk, 2)
            pltpu.make_async_copy(
                zst_hbm.at[pl.ds(0, SBLK), pl.ds(b0, bpw)],
                zs_v.at[slot],
                sem_z,
            ).wait()

            @pl.when(blk + 1 < n_sblk)
            def _():
                pltpu.async_copy(
                    zst_hbm.at[pl.ds((blk + 1) * SBLK, SBLK), pl.ds(b0, bpw)],
                    zs_v.at[1 - slot],
                    sem_z,
                )

            for i in range(SBLK):
                s = blk * SBLK + i
                buf = lax.rem(s, 2)

                @pl.when(s >= 2)
                def _():
                    pltpu.make_async_copy(
                        out_v.at[0],
                        out_hbm.at[0, :, pl.ds(b0, bpw)],
                        sem_w,
                    ).wait()

                def k_body(k, kcarry):
                    z16 = zs_v[slot, i, pl.ds(16 * k, 16)]
                    for d0 in range(0, D, 8):
                        vals = [
                            plsc.load_gather(tt_v.at[d0 + j], [z16])
                            for j in range(8)
                        ]
                        for j in range(8):
                            out_v[buf, d0 + j, pl.ds(16 * k, 16)] = vals[j]
                    return kcarry

                lax.fori_loop(0, bpw // 16, k_body, 0)
                pltpu.async_copy(
                    out_v.at[buf], out_hbm.at[s, :, pl.ds(b0, bpw)], sem_w
                )
            return carry

        lax.fori_loop(0, n_sblk, sblk_body, 0)
        for _ in range(2):
            pltpu.make_async_copy(
                out_v.at[0], out_hbm.at[0, :, pl.ds(b0, bpw)], sem_w
            ).wait()

    return gather


def kernel(zs, rows, cols, row_embedding, col_embedding):
    batch, seq = zs.shape
    rz = jnp.zeros((TBL,), jnp.int32).at[1 : 1 + rows.shape[0]].set(rows)
    cz = jnp.zeros((TBL,), jnp.int32).at[1 : 1 + cols.shape[0]].set(cols)
    table = _build_table(
        rz.reshape(1, TBL), cz.reshape(1, TBL), row_embedding.T, col_embedding.T
    )
    out_t = _make_gather(batch, seq)(table, zs.T)
    return jnp.transpose(out_t, (2, 0, 1))
